# baseline (speedup 1.0000x reference)
import functools

import jax
import jax.numpy as jnp
from jax import lax
from jax.experimental import pallas as pl
from jax.experimental.pallas import tpu as pltpu, tpu_sc as plsc

L = 16


def _sc_spline_call(ctab, idx, w, *, B, C, N, Y, NC, NS, P):
    NW = NC * NS
    chunks = N // (NW * P)
    n_sup = 4
    combos = n_sup * n_sup
    mesh = plsc.VectorSubcoreMesh(core_axis_name="c", subcore_axis_name="s")

    @functools.partial(
        pl.kernel,
        out_type=jax.ShapeDtypeStruct((B, C, N), jnp.float32),
        mesh=mesh,
        scratch_types=[
            pltpu.VMEM((2 * n_sup, P), jnp.int32),
            pltpu.VMEM((2 * n_sup, P), jnp.float32),
            pltpu.VMEM((combos * P,), jnp.int32),
            pltpu.VMEM((combos * P, 8), jnp.float32),
            pltpu.VMEM((C, P), jnp.float32),
            pltpu.SemaphoreType.DMA,
        ],
        compiler_params=pltpu.CompilerParams(
            needs_layout_passes=False, use_tc_tiling_on_sc=False
        ),
    )
    def k(ctab_hbm, idx_hbm, w_hbm, out_hbm, idx_v, w_v, lin_v, g_v, out_v, sem):
        wid = lax.axis_index("s") * NC + lax.axis_index("c")
        lane = lax.iota(jnp.int32, L)
        ch_idx = [jnp.full((L,), e, jnp.int32) for e in range(C)]

        def do_chunk(b, j):
            base = wid * (chunks * P) + j * P
            pltpu.sync_copy(idx_hbm.at[b, :, pl.ds(base, P)], idx_v)
            pltpu.sync_copy(w_hbm.at[b, :, pl.ds(base, P)], w_v)

            def lin_body(t, _):
                s = t * L
                i1v = [idx_v[n_sup + k2, pl.ds(s, L)] for k2 in range(n_sup)]
                for k1 in range(n_sup):
                    r = idx_v[k1, pl.ds(s, L)] * Y
                    for k2 in range(n_sup):
                        lin_v[pl.ds((k1 * n_sup + k2) * P + s, L)] = r + i1v[k2]
                return 0

            lax.fori_loop(0, P // L, lin_body, 0, unroll=False)

            pltpu.async_copy(ctab_hbm.at[b].at[lin_v], g_v, sem).wait()

            def acc_body(t, _):
                s = t * L
                w1v = [w_v[n_sup + k2, pl.ds(s, L)] for k2 in range(n_sup)]
                acc = [jnp.zeros((L,), jnp.float32) for _ in range(C)]
                for k1 in range(n_sup):
                    w0v = w_v[k1, pl.ds(s, L)]
                    for k2 in range(n_sup):
                        wp = w0v * w1v[k2]
                        rows = (k1 * n_sup + k2) * P + s + lane
                        for e in range(C):
                            g = plsc.load_gather(g_v, [rows, ch_idx[e]])
                            acc[e] = acc[e] + wp * g
                for e in range(C):
                    out_v[e, pl.ds(s, L)] = acc[e]
                return 0

            lax.fori_loop(0, P // L, acc_body, 0, unroll=False)
            for e in range(C):
                pltpu.sync_copy(out_v.at[e], out_hbm.at[b, e, pl.ds(base, P)])

        for b in range(B):
            lax.fori_loop(0, chunks, lambda j, _, b=b: (do_chunk(b, j), 0)[1], 0,
                          unroll=False)

    return k(ctab, idx, w)


def kernel(c, weight, index):
    B, C, X, Y = c.shape
    n_sup = weight.shape[0]
    N = X * Y
    ctab = jnp.pad(
        c.transpose(0, 2, 3, 1).reshape(B, N, C), ((0, 0), (0, 0), (0, 8 - C))
    )
    w = weight.transpose(1, 2, 0, 3, 4).reshape(B, 2 * n_sup, N)
    idx = index.transpose(1, 2, 0, 3, 4).reshape(B, 2 * n_sup, N).astype(jnp.int32)
    out = _sc_spline_call(ctab, idx, w, B=B, C=C, N=N, Y=Y, NC=2, NS=16, P=512)
    return out.reshape(B, C, X, Y)


# speedup vs baseline: 1.6405x; 1.6405x over previous
import functools

import jax
import jax.numpy as jnp
from jax import lax
from jax.experimental import pallas as pl
from jax.experimental.pallas import tpu as pltpu, tpu_sc as plsc

L = 16


def _sc_spline_call(cflat, idx, w, *, B, C, N, Y, NC, NS, P):
    NW = NC * NS
    chunks = N // (NW * P)
    n_sup = 4
    combos = n_sup * n_sup
    mesh = plsc.VectorSubcoreMesh(core_axis_name="c", subcore_axis_name="s")

    @functools.partial(
        pl.kernel,
        out_type=jax.ShapeDtypeStruct((B, C, N), jnp.float32),
        mesh=mesh,
        scratch_types=[
            pltpu.VMEM((2 * n_sup, P), jnp.int32),
            pltpu.VMEM((2 * n_sup, P), jnp.float32),
            pltpu.VMEM((combos * P,), jnp.int32),
            pltpu.VMEM((combos * P,), jnp.int32),
            pltpu.VMEM((C, combos * P, 8), jnp.float32),
            pltpu.VMEM((C, P), jnp.float32),
            pltpu.SemaphoreType.DMA,
        ],
        compiler_params=pltpu.CompilerParams(
            needs_layout_passes=False, use_tc_tiling_on_sc=False
        ),
    )
    def k(cflat_hbm, idx_hbm, w_hbm, out_hbm, idx_v, w_v, lin_v, off_v, g_v,
          out_v, sem):
        wid = lax.axis_index("s") * NC + lax.axis_index("c")
        lane = lax.iota(jnp.int32, L)

        def do_chunk(b, j):
            base = wid * (chunks * P) + j * P
            descs = []
            for k in range(n_sup):
                for d in range(2):
                    r = d * n_sup + k
                    descs.append(pltpu.async_copy(
                        idx_hbm.at[k, b, d, pl.ds(base, P)], idx_v.at[r], sem))
                    descs.append(pltpu.async_copy(
                        w_hbm.at[k, b, d, pl.ds(base, P)], w_v.at[r], sem))
            for dsc in descs:
                dsc.wait()

            def lin_body(t, _):
                s = t * L
                i1v = [idx_v[n_sup + k2, pl.ds(s, L)] for k2 in range(n_sup)]
                for k1 in range(n_sup):
                    r = idx_v[k1, pl.ds(s, L)] * Y
                    for k2 in range(n_sup):
                        o = (k1 * n_sup + k2) * P + s
                        lin = r + i1v[k2]
                        lin_v[pl.ds(o, L)] = lin >> 3
                        off_v[pl.ds(o, L)] = lin & 7
                return 0

            lax.fori_loop(0, P // L, lin_body, 0, unroll=False)

            gd = [
                pltpu.async_copy(cflat_hbm.at[b * C + e].at[lin_v], g_v.at[e], sem)
                for e in range(C)
            ]
            for dsc in gd:
                dsc.wait()

            def acc_body(t, _):
                s = t * L
                w1v = [w_v[n_sup + k2, pl.ds(s, L)] for k2 in range(n_sup)]
                acc = [jnp.zeros((L,), jnp.float32) for _ in range(C)]
                for k1 in range(n_sup):
                    w0v = w_v[k1, pl.ds(s, L)]
                    for k2 in range(n_sup):
                        o = (k1 * n_sup + k2) * P + s
                        wp = w0v * w1v[k2]
                        rows = o + lane
                        offv = off_v[pl.ds(o, L)]
                        for e in range(C):
                            g = plsc.load_gather(g_v.at[e], [rows, offv])
                            acc[e] = acc[e] + wp * g
                for e in range(C):
                    out_v[e, pl.ds(s, L)] = acc[e]
                return 0

            lax.fori_loop(0, P // L, acc_body, 0, unroll=False)
            for e in range(C):
                pltpu.sync_copy(out_v.at[e], out_hbm.at[b, e, pl.ds(base, P)])

        for b in range(B):
            lax.fori_loop(0, chunks, lambda j, _, b=b: (do_chunk(b, j), 0)[1], 0,
                          unroll=False)

    return k(cflat, idx, w)


def kernel(c, weight, index):
    B, C, X, Y = c.shape
    n_sup = weight.shape[0]
    N = X * Y
    cflat = c.reshape(B * C, N // 8, 8)
    w = weight.reshape(n_sup, B, 2, N)
    idx = index.reshape(n_sup, B, 2, N)
    out = _sc_spline_call(cflat, idx, w, B=B, C=C, N=N, Y=Y, NC=2, NS=16, P=256)
    return out.reshape(B, C, X, Y)


# speedup vs baseline: 4.5843x; 2.7945x over previous
import functools

import jax
import jax.numpy as jnp
from jax import lax
from jax.experimental import pallas as pl
from jax.experimental.pallas import tpu as pltpu, tpu_sc as plsc

L = 16


def _sc_spline_call(c3, idx, w, *, B, C, N, Y, NC, NS, P):
    chunks = N // (NS * P)
    n_sup = 4
    combos = n_sup * n_sup
    pts_per_tile = N // NS
    Q = 4096
    mesh = plsc.VectorSubcoreMesh(core_axis_name="c", subcore_axis_name="s")

    @functools.partial(
        pl.kernel,
        out_type=jax.ShapeDtypeStruct((B, C, N), jnp.float32),
        mesh=mesh,
        scratch_types=[
            pltpu.VMEM((2 * n_sup, P), jnp.int32),
            pltpu.VMEM((2 * n_sup, P), jnp.float32),
            pltpu.VMEM((combos * P,), jnp.int32),
            pltpu.VMEM((combos * P,), jnp.int32),
            pltpu.VMEM((combos * P, 8), jnp.float32),
            pltpu.VMEM((C, P), jnp.float32),
            pltpu.VMEM((C, Q), jnp.float32),
            pltpu.VMEM_SHARED((N // 4, 8), jnp.float32),
            pltpu.SemaphoreType.DMA,
        ],
        compiler_params=pltpu.CompilerParams(
            needs_layout_passes=False, use_tc_tiling_on_sc=False
        ),
    )
    def k(c3_hbm, idx_hbm, w_hbm, out_hbm, idx_v, w_v, lin_v, off_v, g_v,
          out_v, slab_v, sh, sem):
        cid = lax.axis_index("c")
        sid = lax.axis_index("s")
        lane = lax.iota(jnp.int32, L)
        ch_pat = lane & 1
        pt_pat = lane >> 1
        row_pat = lane >> 3
        col_pat = lane & 7

        def stage_batch(b):
            for q in range(pts_per_tile // Q):
                p0 = sid * pts_per_tile + q * Q
                for e in range(C):
                    pltpu.sync_copy(c3_hbm.at[b, e, pl.ds(p0, Q)], slab_v.at[e])

                def build_body(t, _):
                    v = plsc.load_gather(slab_v, [ch_pat, t * 8 + pt_pat])
                    plsc.store_scatter(g_v, [t * 2 + row_pat, col_pat], v)
                    return 0

                lax.fori_loop(0, C * Q // L, build_body, 0, unroll=False)
                pltpu.sync_copy(g_v.at[pl.ds(0, Q // 4)],
                                sh.at[pl.ds(p0 // 4, Q // 4)])

        def do_chunk(b, j):
            base = sid * (chunks * P) + j * P
            descs = []
            for kk in range(n_sup):
                for d in range(2):
                    r = d * n_sup + kk
                    descs.append(pltpu.async_copy(
                        idx_hbm.at[kk, b, d, pl.ds(base, P)], idx_v.at[r], sem))
                    descs.append(pltpu.async_copy(
                        w_hbm.at[kk, b, d, pl.ds(base, P)], w_v.at[r], sem))
            for dsc in descs:
                dsc.wait()

            def lin_body(t, _):
                s = t * L
                i1v = [idx_v[n_sup + k2, pl.ds(s, L)] for k2 in range(n_sup)]
                for k1 in range(n_sup):
                    r = idx_v[k1, pl.ds(s, L)] * Y
                    for k2 in range(n_sup):
                        o = (k1 * n_sup + k2) * P + s
                        lin = r + i1v[k2]
                        lin_v[pl.ds(o, L)] = lin >> 2
                        off_v[pl.ds(o, L)] = (lin & 3) * 2
                return 0

            lax.fori_loop(0, P // L, lin_body, 0, unroll=False)

            pltpu.async_copy(sh.at[lin_v], g_v, sem).wait()

            def acc_body(t, _):
                s = t * L
                w1v = [w_v[n_sup + k2, pl.ds(s, L)] for k2 in range(n_sup)]
                acc = [jnp.zeros((L,), jnp.float32) for _ in range(C)]
                for k1 in range(n_sup):
                    w0v = w_v[k1, pl.ds(s, L)]
                    for k2 in range(n_sup):
                        o = (k1 * n_sup + k2) * P + s
                        wp = w0v * w1v[k2]
                        rows = o + lane
                        offv = off_v[pl.ds(o, L)]
                        g0 = plsc.load_gather(g_v, [rows, offv])
                        g1 = plsc.load_gather(g_v, [rows, offv + 1])
                        acc[0] = acc[0] + wp * g0
                        acc[1] = acc[1] + wp * g1
                for e in range(C):
                    out_v[e, pl.ds(s, L)] = acc[e]
                return 0

            lax.fori_loop(0, P // L, acc_body, 0, unroll=False)
            for e in range(C):
                pltpu.sync_copy(out_v.at[e], out_hbm.at[b, e, pl.ds(base, P)])

        for bb in range(B // NC):
            b = cid * (B // NC) + bb
            stage_batch(b)
            plsc.subcore_barrier()
            lax.fori_loop(
                0, chunks,
                lambda j, _, b=b: (do_chunk(b, j), 0)[1], 0,
                unroll=False)
            if bb + 1 < B // NC:
                plsc.subcore_barrier()

    return k(c3, idx, w)


def kernel(c, weight, index):
    B, C, X, Y = c.shape
    n_sup = weight.shape[0]
    N = X * Y
    c3 = c.reshape(B, C, N)
    w = weight.reshape(n_sup, B, 2, N)
    idx = index.reshape(n_sup, B, 2, N)
    out = _sc_spline_call(c3, idx, w, B=B, C=C, N=N, Y=Y, NC=2, NS=16, P=256)
    return out.reshape(B, C, X, Y)


# speedup vs baseline: 6.3013x; 1.3745x over previous
import functools

import jax
import jax.numpy as jnp
from jax import lax
from jax.experimental import pallas as pl
from jax.experimental.pallas import tpu as pltpu, tpu_sc as plsc

L = 16


def _sc_spline_call(c3, idx, w, *, B, C, N, Y, NC, NS, P):
    chunks = N // (NS * P)
    n_sup = 4
    combos = n_sup * n_sup
    pts_per_tile = N // NS
    Q = 4096
    mesh = plsc.VectorSubcoreMesh(core_axis_name="c", subcore_axis_name="s")

    @functools.partial(
        pl.kernel,
        out_type=jax.ShapeDtypeStruct((B, C, N), jnp.float32),
        mesh=mesh,
        scratch_types=[
            pltpu.VMEM((2 * n_sup, P), jnp.int32),
            pltpu.VMEM((2, 2 * n_sup, P), jnp.float32),
            pltpu.VMEM((2, combos * P), jnp.int32),
            pltpu.VMEM((2, combos * P), jnp.int32),
            pltpu.VMEM((2, combos * P, 8), jnp.float32),
            pltpu.VMEM((C, P), jnp.float32),
            pltpu.VMEM((C, Q), jnp.float32),
            pltpu.VMEM_SHARED((N // 4, 8), jnp.float32),
            pltpu.SemaphoreType.DMA,
            pltpu.SemaphoreType.DMA,
            pltpu.SemaphoreType.DMA,
        ],
        compiler_params=pltpu.CompilerParams(
            needs_layout_passes=False, use_tc_tiling_on_sc=False
        ),
    )
    def k(c3_hbm, idx_hbm, w_hbm, out_hbm, idx_v, w_v, lin_v, off_v, g_v,
          out_v, slab_v, sh, sem, sem_a, sem_b):
        cid = lax.axis_index("c")
        sid = lax.axis_index("s")
        lane = lax.iota(jnp.int32, L)
        ch_pat = lane & 1
        pt_pat = lane >> 1
        row_pat = lane >> 3
        col_pat = lane & 7
        sems = [sem_a, sem_b]

        def stage_batch(b):
            for q in range(pts_per_tile // Q):
                p0 = sid * pts_per_tile + q * Q
                for e in range(C):
                    pltpu.sync_copy(c3_hbm.at[b, e, pl.ds(p0, Q)], slab_v.at[e])

                def build_body(t, _):
                    v = plsc.load_gather(slab_v, [ch_pat, t * 8 + pt_pat])
                    plsc.store_scatter(g_v.at[0], [t * 2 + row_pat, col_pat], v)
                    return 0

                lax.fori_loop(0, C * Q // L, build_body, 0, unroll=False)
                pltpu.sync_copy(g_v.at[0, pl.ds(0, Q // 4)],
                                sh.at[pl.ds(p0 // 4, Q // 4)])

        def load_and_fire(b, j, u):
            base = sid * (chunks * P) + j * P
            descs = []
            for kk in range(n_sup):
                for d in range(2):
                    r = d * n_sup + kk
                    descs.append(pltpu.async_copy(
                        idx_hbm.at[kk, b, d, pl.ds(base, P)], idx_v.at[r], sem))
                    descs.append(pltpu.async_copy(
                        w_hbm.at[kk, b, d, pl.ds(base, P)], w_v.at[u, r], sem))
            for dsc in descs:
                dsc.wait()

            def lin_body(t, _):
                s = t * L
                i1v = [idx_v[n_sup + k2, pl.ds(s, L)] for k2 in range(n_sup)]
                for k1 in range(n_sup):
                    r = idx_v[k1, pl.ds(s, L)] * Y
                    for k2 in range(n_sup):
                        o = (k1 * n_sup + k2) * P + s
                        lin = r + i1v[k2]
                        lin_v[u, pl.ds(o, L)] = lin >> 2
                        off_v[u, pl.ds(o, L)] = (lin & 3) * 2
                return 0

            lax.fori_loop(0, P // L, lin_body, 0, unroll=False)
            pltpu.async_copy(sh.at[lin_v.at[u]], g_v.at[u], sems[u])

        def acc_and_store(b, j, u):
            base = sid * (chunks * P) + j * P
            pltpu.make_async_copy(sh.at[lin_v.at[u]], g_v.at[u], sems[u]).wait()

            def acc_body(t, _):
                s = t * L
                w1v = [w_v[u, n_sup + k2, pl.ds(s, L)] for k2 in range(n_sup)]
                acc = [jnp.zeros((L,), jnp.float32) for _ in range(C)]
                for k1 in range(n_sup):
                    w0v = w_v[u, k1, pl.ds(s, L)]
                    for k2 in range(n_sup):
                        o = (k1 * n_sup + k2) * P + s
                        wp = w0v * w1v[k2]
                        rows = o + lane
                        offv = off_v[u, pl.ds(o, L)]
                        g0 = plsc.load_gather(g_v.at[u], [rows, offv])
                        g1 = plsc.load_gather(g_v.at[u], [rows, offv + 1])
                        acc[0] = acc[0] + wp * g0
                        acc[1] = acc[1] + wp * g1
                for e in range(C):
                    out_v[e, pl.ds(s, L)] = acc[e]
                return 0

            lax.fori_loop(0, P // L, acc_body, 0, unroll=False)
            for e in range(C):
                pltpu.sync_copy(out_v.at[e], out_hbm.at[b, e, pl.ds(base, P)])

        for bb in range(B // NC):
            b = cid * (B // NC) + bb
            stage_batch(b)
            plsc.subcore_barrier()
            load_and_fire(b, 0, 0)

            def pair_body(t, _):
                load_and_fire(b, 2 * t + 1, 1)
                acc_and_store(b, 2 * t, 0)

                @pl.when(t < chunks // 2 - 1)
                def _():
                    load_and_fire(b, 2 * t + 2, 0)

                acc_and_store(b, 2 * t + 1, 1)
                return 0

            lax.fori_loop(0, chunks // 2, pair_body, 0, unroll=False)
            if bb + 1 < B // NC:
                plsc.subcore_barrier()

    return k(c3, idx, w)


def kernel(c, weight, index):
    B, C, X, Y = c.shape
    n_sup = weight.shape[0]
    N = X * Y
    c3 = c.reshape(B, C, N)
    w = weight.reshape(n_sup, B, 2, N)
    idx = index.reshape(n_sup, B, 2, N)
    out = _sc_spline_call(c3, idx, w, B=B, C=C, N=N, Y=Y, NC=2, NS=16, P=256)
    return out.reshape(B, C, X, Y)


# speedup vs baseline: 6.8431x; 1.0860x over previous
import functools

import jax
import jax.numpy as jnp
from jax import lax
from jax.experimental import pallas as pl
from jax.experimental.pallas import tpu as pltpu, tpu_sc as plsc

L = 16


def _sc_spline_call(c3, idx, w, *, B, C, N, Y, NC, NS, P):
    chunks = N // (NS * P)
    n_sup = 4
    combos = n_sup * n_sup
    pts_per_tile = N // NS
    Q = 4096
    mesh = plsc.VectorSubcoreMesh(core_axis_name="c", subcore_axis_name="s")

    @functools.partial(
        pl.kernel,
        out_type=jax.ShapeDtypeStruct((B, C, N), jnp.float32),
        mesh=mesh,
        scratch_types=[
            pltpu.VMEM((2 * n_sup, P), jnp.int32),
            pltpu.VMEM((2, 2 * n_sup, P), jnp.float32),
            pltpu.VMEM((2, combos * P), jnp.int32),
            pltpu.VMEM((2, n_sup * P), jnp.int32),
            pltpu.VMEM((2, combos * P, 8), jnp.float32),
            pltpu.VMEM((C, P), jnp.float32),
            pltpu.VMEM((C, Q), jnp.float32),
            pltpu.VMEM_SHARED((N // 4, 8), jnp.float32),
            pltpu.SemaphoreType.DMA,
            pltpu.SemaphoreType.DMA,
            pltpu.SemaphoreType.DMA,
        ],
        compiler_params=pltpu.CompilerParams(
            needs_layout_passes=False, use_tc_tiling_on_sc=False
        ),
    )
    def k(c3_hbm, idx_hbm, w_hbm, out_hbm, idx_v, w_v, lin_v, off_v, g_v,
          out_v, slab_v, sh, sem, sem_a, sem_b):
        cid = lax.axis_index("c")
        sid = lax.axis_index("s")
        lane = lax.iota(jnp.int32, L)
        ch_pat = lane & 1
        pt_pat = lane >> 1
        row_pat = lane >> 3
        col_pat = lane & 7
        sems = [sem_a, sem_b]

        def stage_batch(b):
            for q in range(pts_per_tile // Q):
                p0 = sid * pts_per_tile + q * Q
                for e in range(C):
                    pltpu.sync_copy(c3_hbm.at[b, e, pl.ds(p0, Q)], slab_v.at[e])

                def build_body(t, _):
                    v = plsc.load_gather(slab_v, [ch_pat, t * 8 + pt_pat])
                    plsc.store_scatter(g_v.at[0], [t * 2 + row_pat, col_pat], v)
                    return 0

                lax.fori_loop(0, C * Q // L, build_body, 0, unroll=False)
                pltpu.sync_copy(g_v.at[0, pl.ds(0, Q // 4)],
                                sh.at[pl.ds(p0 // 4, Q // 4)])

        def load_and_fire(b, j, u):
            base = sid * (chunks * P) + j * P
            descs = []
            for kk in range(n_sup):
                for d in range(2):
                    r = d * n_sup + kk
                    descs.append(pltpu.async_copy(
                        idx_hbm.at[kk, b, d, pl.ds(base, P)], idx_v.at[r], sem))
                    descs.append(pltpu.async_copy(
                        w_hbm.at[kk, b, d, pl.ds(base, P)], w_v.at[u, r], sem))
            for dsc in descs:
                dsc.wait()

            def lin_body(t, _):
                s = t * L
                i1r = []
                for k2 in range(n_sup):
                    i1 = idx_v[n_sup + k2, pl.ds(s, L)]
                    i1r.append(i1 >> 2)
                    off_v[u, pl.ds(k2 * P + s, L)] = (i1 & 3) * 2
                for k1 in range(n_sup):
                    r = idx_v[k1, pl.ds(s, L)] * (Y // 4)
                    for k2 in range(n_sup):
                        lin_v[u, pl.ds((k1 * n_sup + k2) * P + s, L)] = r + i1r[k2]
                return 0

            lax.fori_loop(0, P // L, lin_body, 0, unroll=False)
            pltpu.async_copy(sh.at[lin_v.at[u]], g_v.at[u], sems[u])

        def acc_and_store(b, j, u):
            base = sid * (chunks * P) + j * P
            pltpu.make_async_copy(sh.at[lin_v.at[u]], g_v.at[u], sems[u]).wait()

            def acc_body(t, _):
                s = t * L
                w1v = [w_v[u, n_sup + k2, pl.ds(s, L)] for k2 in range(n_sup)]
                off0 = [off_v[u, pl.ds(k2 * P + s, L)] for k2 in range(n_sup)]
                off1 = [o + 1 for o in off0]
                acc = [jnp.zeros((L,), jnp.float32) for _ in range(C)]
                for k1 in range(n_sup):
                    w0v = w_v[u, k1, pl.ds(s, L)]
                    for k2 in range(n_sup):
                        wp = w0v * w1v[k2]
                        rows = (k1 * n_sup + k2) * P + s + lane
                        g0 = plsc.load_gather(g_v.at[u], [rows, off0[k2]])
                        g1 = plsc.load_gather(g_v.at[u], [rows, off1[k2]])
                        acc[0] = acc[0] + wp * g0
                        acc[1] = acc[1] + wp * g1
                for e in range(C):
                    out_v[e, pl.ds(s, L)] = acc[e]
                return 0

            lax.fori_loop(0, P // L, acc_body, 0, unroll=False)
            for e in range(C):
                pltpu.sync_copy(out_v.at[e], out_hbm.at[b, e, pl.ds(base, P)])

        for bb in range(B // NC):
            b = cid * (B // NC) + bb
            stage_batch(b)
            plsc.subcore_barrier()
            load_and_fire(b, 0, 0)

            def pair_body(t, _):
                load_and_fire(b, 2 * t + 1, 1)
                acc_and_store(b, 2 * t, 0)

                @pl.when(t < chunks // 2 - 1)
                def _():
                    load_and_fire(b, 2 * t + 2, 0)

                acc_and_store(b, 2 * t + 1, 1)
                return 0

            lax.fori_loop(0, chunks // 2, pair_body, 0, unroll=False)
            if bb + 1 < B // NC:
                plsc.subcore_barrier()

    return k(c3, idx, w)


def kernel(c, weight, index):
    B, C, X, Y = c.shape
    n_sup = weight.shape[0]
    N = X * Y
    c3 = c.reshape(B, C, N)
    w = weight.reshape(n_sup, B, 2, N)
    idx = index.reshape(n_sup, B, 2, N)
    out = _sc_spline_call(c3, idx, w, B=B, C=C, N=N, Y=Y, NC=2, NS=16, P=256)
    return out.reshape(B, C, X, Y)


# speedup vs baseline: 6.9698x; 1.0185x over previous
import functools

import jax
import jax.numpy as jnp
from jax import lax
from jax.experimental import pallas as pl
from jax.experimental.pallas import tpu as pltpu, tpu_sc as plsc

L = 16


def _sc_spline_call(c3, idx, w, *, B, C, N, Y, NC, NS, P):
    chunks = N // (NS * P)
    n_sup = 4
    combos = n_sup * n_sup
    pts_per_tile = N // NS
    Q = 4096
    mesh = plsc.VectorSubcoreMesh(core_axis_name="c", subcore_axis_name="s")

    @functools.partial(
        pl.kernel,
        out_type=jax.ShapeDtypeStruct((B, C, N), jnp.float32),
        mesh=mesh,
        scratch_types=[
            pltpu.VMEM((n_sup, 2, P), jnp.int32),
            pltpu.VMEM((2, n_sup, 2, P), jnp.float32),
            pltpu.VMEM((2, combos * P), jnp.int32),
            pltpu.VMEM((2, n_sup * P), jnp.int32),
            pltpu.VMEM((2, combos * P, 8), jnp.float32),
            pltpu.VMEM((C, P), jnp.float32),
            pltpu.VMEM((C, Q), jnp.float32),
            pltpu.VMEM_SHARED((N // 4, 8), jnp.float32),
            pltpu.SemaphoreType.DMA,
            pltpu.SemaphoreType.DMA,
            pltpu.SemaphoreType.DMA,
        ],
        compiler_params=pltpu.CompilerParams(
            needs_layout_passes=False, use_tc_tiling_on_sc=False
        ),
    )
    def k(c3_hbm, idx_hbm, w_hbm, out_hbm, idx_v, w_v, lin_v, off_v, g_v,
          out_v, slab_v, sh, sem, sem_a, sem_b):
        cid = lax.axis_index("c")
        sid = lax.axis_index("s")
        lane = lax.iota(jnp.int32, L)
        ch_pat = lane & 1
        pt_pat = lane >> 1
        row_pat = lane >> 3
        col_pat = lane & 7
        sems = [sem_a, sem_b]

        def stage_batch(b):
            for q in range(pts_per_tile // Q):
                p0 = sid * pts_per_tile + q * Q
                for e in range(C):
                    pltpu.sync_copy(c3_hbm.at[b, e, pl.ds(p0, Q)], slab_v.at[e])

                def build_body(t, _):
                    v = plsc.load_gather(slab_v, [ch_pat, t * 8 + pt_pat])
                    plsc.store_scatter(g_v.at[0], [t * 2 + row_pat, col_pat], v)
                    return 0

                lax.fori_loop(0, C * Q // L, build_body, 0, unroll=False)
                pltpu.sync_copy(g_v.at[0, pl.ds(0, Q // 4)],
                                sh.at[pl.ds(p0 // 4, Q // 4)])

        def load_and_fire(b, j, u):
            base = sid * (chunks * P) + j * P
            d0 = pltpu.async_copy(
                idx_hbm.at[:, b, :, pl.ds(base, P)], idx_v, sem)
            d1 = pltpu.async_copy(
                w_hbm.at[:, b, :, pl.ds(base, P)], w_v.at[u], sem)
            d0.wait()
            d1.wait()

            def lin_body(t, _):
                s = t * L
                i1r = []
                for k2 in range(n_sup):
                    i1 = idx_v[k2, 1, pl.ds(s, L)]
                    i1r.append(i1 >> 2)
                    off_v[u, pl.ds(k2 * P + s, L)] = (i1 & 3) * 2
                for k1 in range(n_sup):
                    r = idx_v[k1, 0, pl.ds(s, L)] * (Y // 4)
                    for k2 in range(n_sup):
                        lin_v[u, pl.ds((k1 * n_sup + k2) * P + s, L)] = r + i1r[k2]
                return 0

            lax.fori_loop(0, P // L, lin_body, 0, unroll=False)
            pltpu.async_copy(sh.at[lin_v.at[u]], g_v.at[u], sems[u])

        def acc_and_store(b, j, u):
            base = sid * (chunks * P) + j * P
            pltpu.make_async_copy(sh.at[lin_v.at[u]], g_v.at[u], sems[u]).wait()

            def acc_body(t, _):
                s = t * L
                w1v = [w_v[u, k2, 1, pl.ds(s, L)] for k2 in range(n_sup)]
                off0 = [off_v[u, pl.ds(k2 * P + s, L)] for k2 in range(n_sup)]
                off1 = [o + 1 for o in off0]
                acc = [jnp.zeros((L,), jnp.float32) for _ in range(C)]
                for k1 in range(n_sup):
                    w0v = w_v[u, k1, 0, pl.ds(s, L)]
                    for k2 in range(n_sup):
                        wp = w0v * w1v[k2]
                        rows = (k1 * n_sup + k2) * P + s + lane
                        g0 = plsc.load_gather(g_v.at[u], [rows, off0[k2]])
                        g1 = plsc.load_gather(g_v.at[u], [rows, off1[k2]])
                        acc[0] = acc[0] + wp * g0
                        acc[1] = acc[1] + wp * g1
                for e in range(C):
                    out_v[e, pl.ds(s, L)] = acc[e]
                return 0

            lax.fori_loop(0, P // L, acc_body, 0, unroll=False)
            pltpu.sync_copy(out_v, out_hbm.at[b, :, pl.ds(base, P)])

        for bb in range(B // NC):
            b = cid * (B // NC) + bb
            stage_batch(b)
            plsc.subcore_barrier()
            load_and_fire(b, 0, 0)

            def pair_body(t, _):
                load_and_fire(b, 2 * t + 1, 1)
                acc_and_store(b, 2 * t, 0)

                @pl.when(t < chunks // 2 - 1)
                def _():
                    load_and_fire(b, 2 * t + 2, 0)

                acc_and_store(b, 2 * t + 1, 1)
                return 0

            lax.fori_loop(0, chunks // 2, pair_body, 0, unroll=False)
            if bb + 1 < B // NC:
                plsc.subcore_barrier()

    return k(c3, idx, w)


def kernel(c, weight, index):
    B, C, X, Y = c.shape
    n_sup = weight.shape[0]
    N = X * Y
    c3 = c.reshape(B, C, N)
    w = weight.reshape(n_sup, B, 2, N)
    idx = index.reshape(n_sup, B, 2, N)
    out = _sc_spline_call(c3, idx, w, B=B, C=C, N=N, Y=Y, NC=2, NS=16, P=256)
    return out.reshape(B, C, X, Y)
